# initial kernel scaffold (unmeasured)
import functools

import jax
import jax.numpy as jnp
from jax import lax
from jax.experimental import pallas as pl
from jax.experimental.pallas import tpu as pltpu

N_DEV = 32
T_CORR = 64


def kernel(x, A, B, C):
    b, L, d = x.shape
    n = A.shape[1]

    At = A.T
    Bn = jnp.transpose(B, (0, 2, 1))
    Cn = jnp.transpose(C, (0, 2, 1))

    def body(x_ref, at_ref, bn_ref, cn_ref, y_ref, hin_ref, hout_ref,
             send_sem, recv_sem):
        my = lax.axis_index("i")
        dAt = jnp.exp(at_ref[:, :])
        bn = bn_ref[:, :, :]
        cn = cn_ref[:, :, :]

        def step(t, H):
            x_t = x_ref[:, pl.ds(t, 1), :]
            b_t = lax.dynamic_slice_in_dim(bn, t, 1, axis=2)
            c_t = lax.dynamic_slice_in_dim(cn, t, 1, axis=2)
            H = H * dAt[None] + x_t * b_t
            y_ref[:, pl.ds(t, 1), :] = jnp.sum(H * c_t, axis=1, keepdims=True)
            return H

        H = lax.fori_loop(0, L, step, jnp.zeros((b, n, d), jnp.float32))
        hout_ref[:, :, :] = H

        rdma = pltpu.make_async_remote_copy(
            src_ref=hout_ref,
            dst_ref=hin_ref,
            send_sem=send_sem,
            recv_sem=recv_sem,
            device_id=(lax.rem(my + 1, N_DEV),),
            device_id_type=pl.DeviceIdType.MESH,
        )

        @pl.when(my < N_DEV - 1)
        def _():
            rdma.start()
            rdma.wait_send()

        @pl.when(my > 0)
        def _():
            rdma.wait_recv()
            hin = hin_ref[:, :, :]

            def cstep(t, G):
                c_t = lax.dynamic_slice_in_dim(cn, t, 1, axis=2)
                y_ref[:, pl.ds(t, 1), :] = y_ref[:, pl.ds(t, 1), :] + jnp.sum(
                    G * c_t, axis=1, keepdims=True
                )
                return G * dAt[None]

            lax.fori_loop(0, T_CORR, cstep, hin * dAt[None])

    return pl.pallas_call(
        body,
        out_shape=jax.ShapeDtypeStruct((b, L, d), jnp.float32),
        in_specs=[
            pl.BlockSpec(memory_space=pltpu.VMEM),
            pl.BlockSpec(memory_space=pltpu.VMEM),
            pl.BlockSpec(memory_space=pltpu.VMEM),
            pl.BlockSpec(memory_space=pltpu.VMEM),
        ],
        out_specs=pl.BlockSpec(memory_space=pltpu.VMEM),
        scratch_shapes=[
            pltpu.VMEM((b, n, d), jnp.float32),
            pltpu.VMEM((b, n, d), jnp.float32),
            pltpu.SemaphoreType.DMA,
            pltpu.SemaphoreType.DMA,
        ],
        compiler_params=pltpu.CompilerParams(collective_id=0),
    )(x, At, Bn, Cn)


# baseline (device time: 87738 ns/iter reference)
import jax
import jax.numpy as jnp
from jax import lax
from jax.experimental import pallas as pl
from jax.experimental.pallas import tpu as pltpu

N_DEV = 32
T_CORR = 64


def kernel(x, A, B, C):
    b, L, d = x.shape
    n = A.shape[1]

    At = A.T
    Bsq = jnp.transpose(B, (1, 0, 2)).reshape(L, b * n)
    Csq = jnp.transpose(C, (1, 0, 2)).reshape(L, b * n)

    def body(x_ref, at_ref, bsq_ref, csq_ref, y_ref, hin_ref, hout_ref,
             send_sem, recv_sem):
        my = lax.axis_index("i")
        dAt = jnp.exp(at_ref[:, :])
        ones_d = jnp.ones((1, d), jnp.float32)

        def bcast(row):
            m = lax.dot_general(
                row, ones_d, (((0,), (0,)), ((), ())),
                preferred_element_type=jnp.float32,
            )
            return m.reshape(b, n, d)

        def step(t, H):
            x_t = x_ref[:, pl.ds(t, 1), :]
            b_t = bcast(bsq_ref[pl.ds(t, 1), :])
            c_t = bcast(csq_ref[pl.ds(t, 1), :])
            H = H * dAt[None] + x_t * b_t
            y_ref[:, pl.ds(t, 1), :] = jnp.sum(H * c_t, axis=1, keepdims=True)
            return H

        H = lax.fori_loop(0, L, step, jnp.zeros((b, n, d), jnp.float32))
        hout_ref[:, :, :] = H

        rdma = pltpu.make_async_remote_copy(
            src_ref=hout_ref,
            dst_ref=hin_ref,
            send_sem=send_sem,
            recv_sem=recv_sem,
            device_id=(lax.rem(my + 1, N_DEV),),
            device_id_type=pl.DeviceIdType.MESH,
        )

        @pl.when(my < N_DEV - 1)
        def _():
            rdma.start()
            rdma.wait_send()

        @pl.when(my > 0)
        def _():
            rdma.wait_recv()
            hin = hin_ref[:, :, :]

            def cstep(t, G):
                c_t = bcast(csq_ref[pl.ds(t, 1), :])
                y_ref[:, pl.ds(t, 1), :] = y_ref[:, pl.ds(t, 1), :] + jnp.sum(
                    G * c_t, axis=1, keepdims=True
                )
                return G * dAt[None]

            lax.fori_loop(0, T_CORR, cstep, hin * dAt[None])

    return pl.pallas_call(
        body,
        out_shape=jax.ShapeDtypeStruct((b, L, d), jnp.float32),
        in_specs=[
            pl.BlockSpec(memory_space=pltpu.VMEM),
            pl.BlockSpec(memory_space=pltpu.VMEM),
            pl.BlockSpec(memory_space=pltpu.VMEM),
            pl.BlockSpec(memory_space=pltpu.VMEM),
        ],
        out_specs=pl.BlockSpec(memory_space=pltpu.VMEM),
        scratch_shapes=[
            pltpu.VMEM((b, n, d), jnp.float32),
            pltpu.VMEM((b, n, d), jnp.float32),
            pltpu.SemaphoreType.DMA,
            pltpu.SemaphoreType.DMA,
        ],
    )(x, At, Bsq, Csq)


# device time: 11827 ns/iter; 7.4184x vs baseline; 7.4184x over previous
import jax
import jax.numpy as jnp
from jax import lax
from jax.experimental import pallas as pl
from jax.experimental.pallas import tpu as pltpu

N_DEV = 32
K = 8
UNROLL = 32
T_CORR = 32


def kernel(x, A, B, C):
    b, L, d = x.shape
    n = A.shape[1]
    T = L // K

    At = A.T
    Bblk = jnp.transpose(B.reshape(b, K, T, n), (2, 1, 0, 3)).reshape(T, K * b * n)
    Cblk = jnp.transpose(C.reshape(b, K, T, n), (2, 1, 0, 3)).reshape(T, K * b * n)

    def body(x_ref, at_ref, bblk_ref, cblk_ref, y_ref, hin_ref, hout_ref,
             send_sem, recv_sem):
        my = lax.axis_index("i")

        barrier_sem = pltpu.get_barrier_semaphore()

        @pl.when(my > 0)
        def _():
            pl.semaphore_signal(
                barrier_sem, inc=1, device_id=(my - 1,),
                device_id_type=pl.DeviceIdType.MESH,
            )

        @pl.when(my < N_DEV - 1)
        def _():
            pl.semaphore_signal(
                barrier_sem, inc=1, device_id=(my + 1,),
                device_id_type=pl.DeviceIdType.MESH,
            )

        n_nbrs = jnp.where(my > 0, 1, 0) + jnp.where(my < N_DEV - 1, 1, 0)
        pl.semaphore_wait(barrier_sem, n_nbrs)

        dAt = jnp.exp(at_ref[:, :])
        dAT = jnp.exp(at_ref[:, :] * float(T))
        ones_d = jnp.ones((1, d), jnp.float32)

        def bcast(row):
            m = lax.dot_general(
                row, ones_d, (((0,), (0,)), ((), ())),
                preferred_element_type=jnp.float32,
            )
            return m.reshape(K, b, n, d)

        def x_t(t):
            return jnp.stack(
                [x_ref[:, pl.ds(t + kk * T, 1), :] for kk in range(K)], axis=0
            )

        def step1(t, H):
            return H * dAt + x_t(t) * bcast(bblk_ref[pl.ds(t, 1), :])

        Hfin = lax.fori_loop(
            0, T, step1, jnp.zeros((K, b, n, d), jnp.float32), unroll=UNROLL
        )

        def stitch(h0):
            hs = [h0]
            for kk in range(K):
                hs.append(dAT * hs[-1] + Hfin[kk])
            return hs

        hout_ref[:, :, :] = stitch(jnp.zeros((b, n, d), jnp.float32))[K]

        rdma = pltpu.make_async_remote_copy(
            src_ref=hout_ref,
            dst_ref=hin_ref,
            send_sem=send_sem,
            recv_sem=recv_sem,
            device_id=(lax.rem(my + 1, N_DEV),),
            device_id_type=pl.DeviceIdType.MESH,
        )

        @pl.when(my < N_DEV - 1)
        def _():
            rdma.start()

        Hin = jnp.stack(stitch(jnp.zeros((b, n, d), jnp.float32))[:K])

        def step2(t, H):
            H = H * dAt + x_t(t) * bcast(bblk_ref[pl.ds(t, 1), :])
            y = jnp.sum(H * bcast(cblk_ref[pl.ds(t, 1), :]), axis=2)
            for kk in range(K):
                y_ref[:, pl.ds(t + kk * T, 1), :] = y[kk][:, None, :]
            return H

        lax.fori_loop(0, T, step2, Hin, unroll=UNROLL)

        @pl.when(my > 0)
        def _():
            rdma.wait_recv()
            G = hin_ref[:, :, :] * dAt
            for t in range(T_CORR):
                c_t = lax.dot_general(
                    cblk_ref[pl.ds(t, 1), pl.ds(0, b * n)], ones_d,
                    (((0,), (0,)), ((), ())),
                    preferred_element_type=jnp.float32,
                ).reshape(b, n, d)
                y_ref[:, pl.ds(t, 1), :] = y_ref[:, pl.ds(t, 1), :] + jnp.sum(
                    G * c_t, axis=1, keepdims=True
                )
                G = G * dAt

        @pl.when(my < N_DEV - 1)
        def _():
            rdma.wait_send()

    return pl.pallas_call(
        body,
        out_shape=jax.ShapeDtypeStruct((b, L, d), jnp.float32),
        in_specs=[
            pl.BlockSpec(memory_space=pltpu.VMEM),
            pl.BlockSpec(memory_space=pltpu.VMEM),
            pl.BlockSpec(memory_space=pltpu.VMEM),
            pl.BlockSpec(memory_space=pltpu.VMEM),
        ],
        out_specs=pl.BlockSpec(memory_space=pltpu.VMEM),
        scratch_shapes=[
            pltpu.VMEM((b, n, d), jnp.float32),
            pltpu.VMEM((b, n, d), jnp.float32),
            pltpu.SemaphoreType.DMA,
            pltpu.SemaphoreType.DMA,
        ],
        compiler_params=pltpu.CompilerParams(collective_id=0),
    )(x, At, Bblk, Cblk)
